# baseline (speedup 1.0000x reference)
import jax
import jax.numpy as jnp
from jax import lax
from jax.experimental import pallas as pl
from jax.experimental.pallas import tpu as pltpu
from jax.experimental.pallas import tpu_sc as plsc

_INFO = plsc.get_sparse_core_info()
_NC = _INFO.num_cores
_NS = _INFO.num_subcores
_NW = _NC * _NS


def _sc_copy_body(src_hbm, out_hbm, buf):
    wid = lax.axis_index("s") * _NC + lax.axis_index("c")
    n = src_hbm.shape[0] // _NW
    chunk = buf.shape[0]
    base = wid * n
    for i in range(n // chunk):
        off = base + i * chunk
        pltpu.sync_copy(src_hbm.at[pl.ds(off, chunk)], buf)
        pltpu.sync_copy(buf, out_hbm.at[pl.ds(off, chunk)])


def kernel(variable_features, constraint_features, edge_indices, reversed_edge_indices):
    n_var, d = variable_features.shape
    flat = variable_features.reshape(-1)
    per_worker = flat.shape[0] // _NW
    chunk = per_worker
    while chunk * 4 > 400_000:
        chunk //= 2
    mesh = plsc.VectorSubcoreMesh(core_axis_name="c", subcore_axis_name="s")
    out = pl.kernel(
        _sc_copy_body,
        out_type=jax.ShapeDtypeStruct(flat.shape, flat.dtype),
        mesh=mesh,
        scratch_types=[pltpu.VMEM((chunk,), jnp.float32)],
    )(flat)
    return out.reshape(n_var, d)
